# baseline (speedup 1.0000x reference)
import functools

import jax
import jax.numpy as jnp
from jax import lax
from jax.experimental import pallas as pl
from jax.experimental.pallas import tpu as pltpu
from jax.experimental.pallas import tpu_sc as plsc

B = 16384
DIM = 32
VOCAB = 1000000
RPR = 4
VOCAB_PACKED = VOCAB // RPR
NC = 2
NS = 16
NW = NC * NS
BPW = B // NW
L = 16
CHUNK = 128
NCHUNK = BPW // CHUNK
NROUND = 2
RND = BPW // NROUND
GPR = RND // L
CPR = NCHUNK // NROUND


def _body(pairs_hbm, emb_hbm, ctx_hbm, out_hbm,
          pv, crow, trow, csub, tsub, erows, crows, outv, sem):
    wid = lax.axis_index("s") * NC + lax.axis_index("c")
    base = wid * BPW

    pltpu.sync_copy(pairs_hbm.at[pl.ds(2 * base, 2 * BPW)], pv)

    iota = lax.iota(jnp.int32, L)
    for g in range(BPW // L):
        flat = 2 * (g * L + iota)
        cid = plsc.load_gather(pv, [flat])
        tid = plsc.load_gather(pv, [flat + 1])
        j, c0 = (g * L) // CHUNK, (g * L) % CHUNK
        crow[j, pl.ds(c0, L)] = lax.shift_right_logical(cid, 2)
        trow[j, pl.ds(c0, L)] = lax.shift_right_logical(tid, 2)
        csub[pl.ds(g * L, L)] = lax.shift_left(jnp.bitwise_and(cid, 3), 5)
        tsub[pl.ds(g * L, L)] = lax.shift_left(jnp.bitwise_and(tid, 3), 5)

    for r in range(NROUND):
        copies = []
        for jj in range(CPR):
            j = r * CPR + jj
            dst_e = erows.at[pl.ds(jj * CHUNK, CHUNK), :]
            dst_c = crows.at[pl.ds(jj * CHUNK, CHUNK), :]
            copies.append(pltpu.async_copy(emb_hbm.at[crow.at[j]], dst_e, sem))
            copies.append(pltpu.async_copy(ctx_hbm.at[trow.at[j]], dst_c, sem))
        for cp in copies:
            cp.wait()

        def group(g, _, r=r):
            lrow = g * L + iota
            grow = r * RND + g * L + iota
            ccol = plsc.load_gather(csub, [grow])
            tcol = plsc.load_gather(tsub, [grow])
            acc = jnp.zeros((L,), jnp.float32)
            for d in range(DIM):
                a = plsc.load_gather(erows, [lrow, ccol + d])
                b = plsc.load_gather(crows, [lrow, tcol + d])
                acc = acc + a * b
            y = 1.0 / (1.0 + jnp.exp(-acc))
            plsc.store_scatter(outv, [grow], y)
            return 0

        lax.fori_loop(0, GPR, group, 0)

    pltpu.sync_copy(outv, out_hbm.at[pl.ds(base, BPW)])


@jax.jit
def _skipgram(pairs, emb_w, ctx_w):
    mesh = plsc.VectorSubcoreMesh(core_axis_name="c", subcore_axis_name="s")
    k = pl.kernel(
        _body,
        out_type=jax.ShapeDtypeStruct((B,), jnp.float32),
        mesh=mesh,
        compiler_params=pltpu.CompilerParams(needs_layout_passes=False),
        scratch_types=[
            pltpu.VMEM((2 * BPW,), jnp.int32),
            pltpu.VMEM((NCHUNK, CHUNK), jnp.int32),
            pltpu.VMEM((NCHUNK, CHUNK), jnp.int32),
            pltpu.VMEM((BPW,), jnp.int32),
            pltpu.VMEM((BPW,), jnp.int32),
            pltpu.VMEM((RND, RPR * DIM), jnp.float32),
            pltpu.VMEM((RND, RPR * DIM), jnp.float32),
            pltpu.VMEM((BPW,), jnp.float32),
            pltpu.SemaphoreType.DMA,
        ],
    )
    return k(pairs, emb_w, ctx_w)


def kernel(pairs, emb_w, ctx_w):
    return _skipgram(
        pairs.astype(jnp.int32).reshape(-1),
        emb_w.reshape(VOCAB_PACKED, RPR * DIM),
        ctx_w.reshape(VOCAB_PACKED, RPR * DIM),
    )


# speedup vs baseline: 1.4555x; 1.4555x over previous
import jax
import jax.numpy as jnp
from jax import lax
from jax.experimental import pallas as pl
from jax.experimental.pallas import tpu as pltpu
from jax.experimental.pallas import tpu_sc as plsc

B = 16384
DIM = 32
NC = 2
NS = 16
NW = NC * NS
BPW = B // NW
L = 16
RP = 64
NROUND = BPW // RP
GPR = RP // L


def _body(pairs_hbm, emb_hbm, ctx_hbm, out_hbm, pv, erows, crows, outv, sem):
    wid = lax.axis_index("s") * NC + lax.axis_index("c")
    base = wid * BPW

    pltpu.sync_copy(pairs_hbm.at[pl.ds(2 * base, 2 * BPW)], pv)

    iota = lax.iota(jnp.int32, L)
    dcols = [jnp.full((L,), d, jnp.int32) for d in range(DIM)]

    def fire(g, _):
        ids0 = pv[pl.ds(2 * L * g, L)]
        ids1 = pv[pl.ds(2 * L * g + L, L)]
        for lane in range(L):
            ids = ids0 if lane < L // 2 else ids1
            c = ids[(2 * lane) % L]
            t = ids[(2 * lane + 1) % L]
            row = (g % GPR) * L + lane
            pltpu.async_copy(
                emb_hbm.at[pl.ds(c, 1), :], erows.at[pl.ds(row, 1), :], sem)
            pltpu.async_copy(
                ctx_hbm.at[pl.ds(t, 1), :], crows.at[pl.ds(row, 1), :], sem)
        return 0

    def group(g, _):
        rows = (g % GPR) * L + iota
        acc = jnp.zeros((L,), jnp.float32)
        for d in range(DIM):
            a = plsc.load_gather(erows, [rows, dcols[d]])
            b = plsc.load_gather(crows, [rows, dcols[d]])
            acc = acc + a * b
        y = 1.0 / (1.0 + jnp.exp(-acc))
        plsc.store_scatter(outv, [g * L + iota], y)
        return 0

    for r in range(NROUND):
        lax.fori_loop(r * GPR, (r + 1) * GPR, fire, 0)
        pltpu.make_async_copy(emb_hbm.at[pl.ds(0, RP), :], erows, sem).wait()
        pltpu.make_async_copy(ctx_hbm.at[pl.ds(0, RP), :], crows, sem).wait()
        lax.fori_loop(r * GPR, (r + 1) * GPR, group, 0)

    pltpu.sync_copy(outv, out_hbm.at[pl.ds(base, BPW)])


@jax.jit
def _skipgram(pairs, emb_w, ctx_w):
    mesh = plsc.VectorSubcoreMesh(core_axis_name="c", subcore_axis_name="s")
    k = pl.kernel(
        _body,
        out_type=jax.ShapeDtypeStruct((B,), jnp.float32),
        mesh=mesh,
        compiler_params=pltpu.CompilerParams(needs_layout_passes=False),
        scratch_types=[
            pltpu.VMEM((2 * BPW,), jnp.int32),
            pltpu.VMEM((RP, DIM), jnp.float32),
            pltpu.VMEM((RP, DIM), jnp.float32),
            pltpu.VMEM((BPW,), jnp.float32),
            pltpu.SemaphoreType.DMA,
        ],
    )
    return k(pairs, emb_w, ctx_w)


def kernel(pairs, emb_w, ctx_w):
    return _skipgram(pairs.astype(jnp.int32).reshape(-1), emb_w, ctx_w)
